# baseline (speedup 1.0000x reference)
import functools

import jax
import jax.numpy as jnp
from jax import lax
from jax.experimental import pallas as pl
from jax.experimental.pallas import tpu as pltpu
from jax.experimental.pallas import tpu_sc as plsc

_VOCAB = 1000000
_EMBED_DIM = 64
_BATCH = 16384
_HIST = 50
_B = _BATCH * _HIST

_info = plsc.get_sparse_core_info()
_NC = _info.num_cores
_NS = _info.num_subcores
_NW = _NC * _NS
_BPW = _B // _NW
_CH = 256
_NCHUNK = _BPW // _CH
_NBUF = 4
assert _NCHUNK % _NBUF == 0

_mesh = plsc.VectorSubcoreMesh(core_axis_name="c", subcore_axis_name="s")


@functools.partial(
    pl.kernel,
    mesh=_mesh,
    out_type=jax.ShapeDtypeStruct((_B, _EMBED_DIM), jnp.float32),
    scratch_types=[
        pltpu.VMEM((_NCHUNK, _CH), jnp.int32),
        [pltpu.VMEM((_CH, _EMBED_DIM), jnp.float32)] * _NBUF,
        [pltpu.SemaphoreType.DMA] * _NBUF,
        [pltpu.SemaphoreType.DMA] * _NBUF,
    ],
    compiler_params=pltpu.CompilerParams(use_tc_tiling_on_sc=False),
)
def _gather_kernel(idx_hbm, table_hbm, out_hbm, idx_v, rows, gsems, wsems):
    wid = lax.axis_index("s") * _NC + lax.axis_index("c")
    base = wid * _BPW

    table2 = table_hbm
    out2 = out_hbm
    idx2 = idx_hbm

    pltpu.sync_copy(idx2.at[pl.ds(wid * _NCHUNK, _NCHUNK)], idx_v)

    def g_start(c, b):
        pltpu.async_copy(table2.at[idx_v.at[c]], rows[b], gsems[b])

    def g_wait(c, b):
        pltpu.make_async_copy(table2.at[idx_v.at[c]], rows[b],
                              gsems[b]).wait()

    def w_start(c, b):
        pltpu.async_copy(rows[b], out2.at[pl.ds(base + c * _CH, _CH)],
                         wsems[b])

    def w_wait(c, b):
        pltpu.make_async_copy(rows[b], out2.at[pl.ds(base + c * _CH, _CH)],
                              wsems[b]).wait()

    for b in range(_NBUF - 1):
        g_start(b, b)

    def body(g, _):
        for b in range(_NBUF):
            c = g * _NBUF + b
            g_wait(c, b)
            w_start(c, b)
            n = c + _NBUF - 1
            nb = (b - 1) % _NBUF

            @pl.when(n < _NCHUNK)
            def _(c=c, n=n, nb=nb):
                @pl.when(n >= _NBUF)
                def _():
                    w_wait(n - _NBUF, nb)
                g_start(n, nb)
        return 0

    lax.fori_loop(0, _NCHUNK // _NBUF, body, 0)
    for b in range(_NBUF):
        c = _NCHUNK - _NBUF + b
        w_wait(c, b)


def kernel(input_ids, table):
    idx = input_ids.reshape(_B // _CH, _CH).astype(jnp.int32)
    out = _gather_kernel(idx, table)
    return out.reshape(_BATCH, _HIST, _EMBED_DIM)


# speedup vs baseline: 1.2999x; 1.2999x over previous
import functools

import jax
import jax.numpy as jnp
from jax import lax
from jax.experimental import pallas as pl
from jax.experimental.pallas import tpu as pltpu
from jax.experimental.pallas import tpu_sc as plsc

_VOCAB = 1000000
_EMBED_DIM = 64
_BATCH = 16384
_HIST = 50
_HPAD = 56
_EPAD = 128

_info = plsc.get_sparse_core_info()
_NC = _info.num_cores
_NS = _info.num_subcores
_NW = _NC * _NS
_BPW = _BATCH // _NW
_NBUF = 4
assert _BPW % _NBUF == 0

_mesh = plsc.VectorSubcoreMesh(core_axis_name="c", subcore_axis_name="s")


@functools.partial(
    pl.kernel,
    mesh=_mesh,
    out_type=jax.ShapeDtypeStruct((_BATCH, _HPAD, _EPAD), jnp.float32),
    scratch_types=[
        pltpu.VMEM((_BPW, _HIST), jnp.int32),
        [pltpu.VMEM((_HIST, _EMBED_DIM), jnp.float32)] * _NBUF,
        [pltpu.SemaphoreType.DMA] * _NBUF,
        [pltpu.SemaphoreType.DMA] * _NBUF,
    ],
    compiler_params=pltpu.CompilerParams(use_tc_tiling_on_sc=False),
)
def _gather_kernel(idx_hbm, table_hbm, out_hbm, idx_v, rows, gsems, wsems):
    wid = lax.axis_index("s") * _NC + lax.axis_index("c")
    base = wid * _BPW

    pltpu.sync_copy(idx_hbm.at[pl.ds(base, _BPW)], idx_v)

    def g_start(c, b):
        pltpu.async_copy(table_hbm.at[idx_v.at[c]], rows[b], gsems[b])

    def g_wait(c, b):
        pltpu.make_async_copy(table_hbm.at[idx_v.at[c]], rows[b],
                              gsems[b]).wait()

    def w_start(c, b):
        pltpu.async_copy(
            rows[b],
            out_hbm.at[base + c, pl.ds(0, _HIST), pl.ds(0, _EMBED_DIM)],
            wsems[b])

    def w_wait(c, b):
        pltpu.make_async_copy(
            rows[b],
            out_hbm.at[base + c, pl.ds(0, _HIST), pl.ds(0, _EMBED_DIM)],
            wsems[b]).wait()

    for b in range(_NBUF - 1):
        g_start(b, b)

    def body(g, _):
        for b in range(_NBUF):
            c = g * _NBUF + b
            g_wait(c, b)
            w_start(c, b)
            n = c + _NBUF - 1
            nb = (b - 1) % _NBUF

            @pl.when(n < _BPW)
            def _(c=c, n=n, nb=nb):
                @pl.when(n >= _NBUF)
                def _():
                    w_wait(n - _NBUF, nb)
                g_start(n, nb)
        return 0

    lax.fori_loop(0, _BPW // _NBUF, body, 0)
    for b in range(_NBUF):
        c = _BPW - _NBUF + b
        w_wait(c, b)


def kernel(input_ids, table):
    idx = input_ids.astype(jnp.int32)
    out = _gather_kernel(idx, table)
    return out[:, :_HIST, :_EMBED_DIM]


# speedup vs baseline: 1.3491x; 1.0379x over previous
import functools

import jax
import jax.numpy as jnp
from jax import lax
from jax.experimental import pallas as pl
from jax.experimental.pallas import tpu as pltpu
from jax.experimental.pallas import tpu_sc as plsc

_VOCAB = 1000000
_EMBED_DIM = 64
_BATCH = 16384
_HIST = 50
_HPAD = 56
_EPAD = 128

_info = plsc.get_sparse_core_info()
_NC = _info.num_cores
_NS = _info.num_subcores
_NW = _NC * _NS
_BPW = _BATCH // _NW
_CB = 4
_NCHUNK = _BPW // _CB
_CR = _CB * _HIST
_NBUF = 4
assert _NCHUNK % _NBUF == 0

_mesh = plsc.VectorSubcoreMesh(core_axis_name="c", subcore_axis_name="s")


@functools.partial(
    pl.kernel,
    mesh=_mesh,
    out_type=jax.ShapeDtypeStruct((_BATCH, _HPAD, _EPAD), jnp.float32),
    scratch_types=[
        pltpu.VMEM((_NCHUNK, _CR), jnp.int32),
        [pltpu.VMEM((_CR, _EMBED_DIM), jnp.float32)] * _NBUF,
        [pltpu.SemaphoreType.DMA] * _NBUF,
        [pltpu.SemaphoreType.DMA] * _NBUF,
    ],
    compiler_params=pltpu.CompilerParams(use_tc_tiling_on_sc=False),
)
def _gather_kernel(idx_hbm, table_hbm, out_hbm, idx_v, rows, gsems, wsems):
    wid = lax.axis_index("s") * _NC + lax.axis_index("c")
    base = wid * _BPW

    pltpu.sync_copy(idx_hbm.at[pl.ds(wid * _NCHUNK, _NCHUNK)], idx_v)

    def g_start(c, b):
        pltpu.async_copy(table_hbm.at[idx_v.at[c]], rows[b], gsems[b])

    def g_wait(c, b):
        pltpu.make_async_copy(table_hbm.at[idx_v.at[c]], rows[b],
                              gsems[b]).wait()

    def _w_copies(c, b):
        for k in range(_CB):
            yield pltpu.make_async_copy(
                rows[b].at[pl.ds(k * _HIST, _HIST)],
                out_hbm.at[base + c * _CB + k,
                           pl.ds(0, _HIST), pl.ds(0, _EMBED_DIM)],
                wsems[b])

    def w_start(c, b):
        for cp in _w_copies(c, b):
            cp.start()

    def w_wait(c, b):
        for cp in _w_copies(c, b):
            cp.wait()

    for b in range(_NBUF - 1):
        g_start(b, b)

    def body(g, _):
        for b in range(_NBUF):
            c = g * _NBUF + b
            g_wait(c, b)
            w_start(c, b)
            n = c + _NBUF - 1
            nb = (b - 1) % _NBUF

            @pl.when(n < _NCHUNK)
            def _(c=c, n=n, nb=nb):
                @pl.when(n >= _NBUF)
                def _():
                    w_wait(n - _NBUF, nb)
                g_start(n, nb)
        return 0

    lax.fori_loop(0, _NCHUNK // _NBUF, body, 0)
    for b in range(_NBUF):
        c = _NCHUNK - _NBUF + b
        w_wait(c, b)


def kernel(input_ids, table):
    idx = input_ids.astype(jnp.int32).reshape(_BATCH * _HIST // _CR, _CR)
    out = _gather_kernel(idx, table)
    return out[:, :_HIST, :_EMBED_DIM]


# speedup vs baseline: 1.3498x; 1.0005x over previous
import functools

import jax
import jax.numpy as jnp
from jax import lax
from jax.experimental import pallas as pl
from jax.experimental.pallas import tpu as pltpu
from jax.experimental.pallas import tpu_sc as plsc

_VOCAB = 1000000
_EMBED_DIM = 64
_BATCH = 16384
_HIST = 50
_HPAD = 56
_EPAD = 128

_info = plsc.get_sparse_core_info()
_NC = _info.num_cores
_NS = _info.num_subcores
_NW = _NC * _NS
_BPW = _BATCH // _NW
_CB = 4
_NCHUNK = _BPW // _CB
_CR = _CB * _HIST
_NBUF = 8
assert _NCHUNK % _NBUF == 0

_mesh = plsc.VectorSubcoreMesh(core_axis_name="c", subcore_axis_name="s")

@functools.partial(
    pl.kernel,
    mesh=_mesh,
    out_type=jax.ShapeDtypeStruct((_BATCH, _HPAD, _EPAD), jnp.float32),
    scratch_types=[
        pltpu.VMEM((_NCHUNK, _CR), jnp.int32),
        [pltpu.VMEM((_CR, _EMBED_DIM), jnp.float32)] * _NBUF,
        [pltpu.SemaphoreType.DMA] * _NBUF,
        [pltpu.SemaphoreType.DMA] * _NBUF,
    ],
    compiler_params=pltpu.CompilerParams(use_tc_tiling_on_sc=False),
)
def _gather_kernel(idx_hbm, table_hbm, out_hbm, idx_v, rows, gsems, wsems):
    wid = lax.axis_index("s") * _NC + lax.axis_index("c")
    base = wid * _BPW

    pltpu.sync_copy(idx_hbm.at[pl.ds(wid * _NCHUNK, _NCHUNK)], idx_v)

    def g_start(c, b):
        pltpu.async_copy(table_hbm.at[idx_v.at[c]], rows[b], gsems[b])

    def g_wait(c, b):
        pltpu.make_async_copy(table_hbm.at[idx_v.at[c]], rows[b],
                              gsems[b]).wait()

    def _w_copies(c, b):
        for k in range(_CB):
            yield pltpu.make_async_copy(
                rows[b].at[pl.ds(k * _HIST, _HIST)],
                out_hbm.at[base + c * _CB + k,
                           pl.ds(0, _HIST), pl.ds(0, _EMBED_DIM)],
                wsems[b])

    def w_start(c, b):
        for cp in _w_copies(c, b):
            cp.start()

    def w_wait(c, b):
        for cp in _w_copies(c, b):
            cp.wait()

    for b in range(_NBUF - 1):
        g_start(b, b)

    def body(g, _):
        for b in range(_NBUF):
            c = g * _NBUF + b
            g_wait(c, b)
            w_start(c, b)
            n = c + _NBUF - 1
            nb = (b - 1) % _NBUF

            @pl.when(n < _NCHUNK)
            def _(c=c, n=n, nb=nb):
                @pl.when(n >= _NBUF)
                def _():
                    w_wait(n - _NBUF, nb)
                g_start(n, nb)
        return 0

    lax.fori_loop(0, _NCHUNK // _NBUF, body, 0)
    for b in range(_NBUF):
        c = _NCHUNK - _NBUF + b
        w_wait(c, b)


def kernel(input_ids, table):
    idx = input_ids.astype(jnp.int32).reshape(_BATCH * _HIST // _CR, _CR)
    out = _gather_kernel(idx, table)
    return out[:, :_HIST, :_EMBED_DIM]
